# baseline (speedup 1.0000x reference)
import jax
import jax.numpy as jnp
from jax.experimental import pallas as pl
from jax.experimental.pallas import tpu as pltpu

_F32 = jnp.float32
_BF16 = jnp.bfloat16


def _bdot(a, b):
    return jnp.dot(a.astype(_BF16), b.astype(_BF16),
                   preferred_element_type=_F32)


def _body(g_ref, x_ref, orig_ref, origf_ref, W_ref, al_ref, w0T_ref, w1T_ref,
          cb_ref, sw_ref, D_ref, out_ref, xw_scr, e_scr, s_scr, P_scr,
          Bm_scr):
    b = pl.program_id(0)
    t = pl.program_id(1)
    Bg = pl.num_programs(0)
    S = pl.num_programs(1)
    N, l_dim = e_scr.shape
    K = P_scr.shape[0]
    TILE = g_ref.shape[1]

    @pl.when(t == 0)
    def _():
        xw_scr[...] = _bdot(x_ref[0], W_ref[...]).astype(_BF16)

    gt = g_ref[0]
    deg = jnp.sum(gt, axis=1, keepdims=True)
    h = jnp.dot((gt / deg).astype(_BF16), xw_scr[...],
                preferred_element_type=_F32)
    hn = jnp.minimum(h, 0.0)
    u = jnp.exp(hn)
    em1 = jnp.where(u == 1.0, hn, (u - 1.0) * hn / jnp.log(u))
    e_scr[pl.ds(t * TILE, TILE), :] = jnp.where(h > 0, h, em1)

    @pl.when(t == S - 1)
    def _():
        e = e_scr[...]
        n_el = jnp.float32(N * l_dim)
        mu = jnp.sum(e) / n_el
        ec = e - mu
        var = jnp.sum(ec * ec) / n_el
        sig = jnp.sqrt(var + 1e-5)

        s_scr[...] = _bdot((e - mu) / sig, sw_ref[...]).reshape(N // 128, 128)

        flat_iota = (jax.lax.broadcasted_iota(jnp.int32, (N // 128, 128), 0)
                     * 128
                     + jax.lax.broadcasted_iota(jnp.int32, (N // 128, 128), 1))
        row_iota = jax.lax.broadcasted_iota(jnp.int32, (1, N), 1)

        def tk_body(i, _):
            s = s_scr[...]
            m = jnp.max(s, keepdims=True)
            idx = jnp.min(jnp.where(s >= m, flat_iota, N), keepdims=True)
            P_scr[pl.ds(i, 1), :] = (row_iota == idx).astype(_F32)
            s_scr[...] = jnp.where(flat_iota == idx, -jnp.inf, s)
            return 0

        jax.lax.fori_loop(0, K, tk_body, 0)

        picked = jnp.dot(P_scr[...], e, preferred_element_type=_F32,
                         precision=jax.lax.Precision.HIGHEST)
        sub = (picked - mu) / sig

        w0T = w0T_ref[...]
        w1T = w1T_ref[...]
        cb = cb_ref[...]

        def cnn(z):
            y0 = _bdot(z, w0T)
            y1 = _bdot(z, w1T)
            y = jnp.maximum(y0[:-1] + y1[1:] + cb, 0.0)
            return jnp.max(y, axis=0)

        a0 = al_ref[0, 0]
        a1 = al_ref[0, 1]
        a2 = al_ref[0, 2]
        am = jnp.maximum(a0, jnp.maximum(a1, a2))
        e0 = jnp.exp(a0 - am)
        e1 = jnp.exp(a1 - am)
        e2 = jnp.exp(a2 - am)
        inv_se = 1.0 / (e0 + e1 + e2)

        bvec = (e0 * cnn(sub) + e1 * cnn(jnp.maximum(sub, 0.0))
                + e2 * cnn(jnp.tanh(sub))) * inv_se
        Bm_scr[pl.ds(b, 1), :] = bvec.reshape(1, l_dim)

        @pl.when(b == Bg - 1)
        def _():
            Md = _bdot(Bm_scr[...], D_ref[...])
            out_ref[0, :] = jnp.sum(orig_ref[...] * Md, axis=1)
            out_ref[1, :] = jnp.sum(origf_ref[...] * Md, axis=1)


def kernel(g, x, original, W_gcn, alphas, conv_w, conv_b, score_w, disc_w):
    Bg, N, _ = g.shape
    l_dim = W_gcn.shape[1]
    K = 64
    TILE = 512
    S = N // TILE

    w0T = conv_w[:, :, 0].T
    w1T = conv_w[:, :, 1].T
    cb_row = conv_b.reshape(1, l_dim)
    sw_col = score_w.reshape(l_dim, 1)
    orig_f = original[::-1]

    const = lambda *s: pl.BlockSpec(s, lambda b, t: (0,) * len(s))
    out2 = pl.pallas_call(
        _body,
        grid=(Bg, S),
        in_specs=[
            pl.BlockSpec((1, TILE, N), lambda b, t: (b, t, 0)),
            pl.BlockSpec((1, N, x.shape[2]), lambda b, t: (b, 0, 0)),
            const(Bg, l_dim),
            const(Bg, l_dim),
            const(W_gcn.shape[0], l_dim),
            const(1, alphas.shape[1]),
            const(l_dim, l_dim),
            const(l_dim, l_dim),
            const(1, l_dim),
            const(l_dim, 1),
            const(l_dim, l_dim),
        ],
        out_specs=pl.BlockSpec((2, Bg), lambda b, t: (0, 0)),
        out_shape=jax.ShapeDtypeStruct((2, Bg), _F32),
        scratch_shapes=[
            pltpu.VMEM((N, l_dim), _BF16),
            pltpu.VMEM((N, l_dim), _F32),
            pltpu.VMEM((N // 128, 128), _F32),
            pltpu.VMEM((K, N), _F32),
            pltpu.VMEM((Bg, l_dim), _F32),
        ],
        compiler_params=pltpu.CompilerParams(
            dimension_semantics=("arbitrary", "arbitrary"),
        ),
    )(g, x, original, orig_f, W_gcn, alphas, w0T, w1T, cb_row, sw_col, disc_w)

    return jnp.concatenate([out2[0], out2[1][::-1]], axis=0)


# speedup vs baseline: 1.0150x; 1.0150x over previous
import jax
import jax.numpy as jnp
from jax.experimental import pallas as pl
from jax.experimental.pallas import tpu as pltpu

_F32 = jnp.float32
_BF16 = jnp.bfloat16


def _bdot(a, b):
    return jnp.dot(a.astype(_BF16), b.astype(_BF16),
                   preferred_element_type=_F32)


def _body(g_ref, x_ref, orig_ref, origf_ref, W_ref, al_ref, w0T_ref, w1T_ref,
          cb_ref, sw_ref, D_ref, out_ref, xw_scr, e_scr, s_scr, P_scr,
          Bm_scr):
    b = pl.program_id(0)
    t = pl.program_id(1)
    Bg = pl.num_programs(0)
    S = pl.num_programs(1)
    N, l_dim = e_scr.shape
    K = P_scr.shape[0]
    TILE = g_ref.shape[1]

    @pl.when(t == 0)
    def _():
        xw_scr[...] = _bdot(x_ref[0], W_ref[...]).astype(_BF16)

    gt = g_ref[0]
    deg = jnp.sum(gt, axis=1, keepdims=True)
    h = jnp.dot((gt / deg).astype(_BF16), xw_scr[...],
                preferred_element_type=_F32)
    hn = jnp.minimum(h, 0.0)
    u = jnp.exp(hn)
    em1 = jnp.where(u == 1.0, hn, (u - 1.0) * hn / jnp.log(u))
    e_scr[pl.ds(t * TILE, TILE), :] = jnp.where(h > 0, h, em1)

    @pl.when(t == S - 1)
    def _():
        e = e_scr[...]
        n_el = jnp.float32(N * l_dim)
        mu = jnp.sum(e) / n_el
        ec = e - mu
        var = jnp.sum(ec * ec) / n_el
        sig = jnp.sqrt(var + 1e-5)

        s_scr[...] = _bdot((e - mu) / sig, sw_ref[...]).reshape(N // 128, 128)

        flat_iota = (jax.lax.broadcasted_iota(jnp.int32, (N // 128, 128), 0)
                     * 128
                     + jax.lax.broadcasted_iota(jnp.int32, (N // 128, 128), 1))
        row_iota = jax.lax.broadcasted_iota(jnp.int32, (1, N), 1)

        def tk_body(i, s):
            m = jnp.max(s, keepdims=True)
            idx = jnp.min(jnp.where(s >= m, flat_iota, N), keepdims=True)
            P_scr[pl.ds(i, 1), :] = (row_iota == idx).astype(_F32)
            return jnp.where(flat_iota == idx, -jnp.inf, s)

        jax.lax.fori_loop(0, K, tk_body, s_scr[...])

        picked = jnp.dot(P_scr[...], e, preferred_element_type=_F32,
                         precision=jax.lax.Precision.HIGHEST)
        sub = (picked - mu) / sig

        w0T = w0T_ref[...]
        w1T = w1T_ref[...]
        cb = cb_ref[...]

        def cnn(z):
            y0 = _bdot(z, w0T)
            y1 = _bdot(z, w1T)
            y = jnp.maximum(y0[:-1] + y1[1:] + cb, 0.0)
            return jnp.max(y, axis=0)

        a0 = al_ref[0, 0]
        a1 = al_ref[0, 1]
        a2 = al_ref[0, 2]
        am = jnp.maximum(a0, jnp.maximum(a1, a2))
        e0 = jnp.exp(a0 - am)
        e1 = jnp.exp(a1 - am)
        e2 = jnp.exp(a2 - am)
        inv_se = 1.0 / (e0 + e1 + e2)

        bvec = (e0 * cnn(sub) + e1 * cnn(jnp.maximum(sub, 0.0))
                + e2 * cnn(jnp.tanh(sub))) * inv_se
        Bm_scr[pl.ds(b, 1), :] = bvec.reshape(1, l_dim)

        @pl.when(b == Bg - 1)
        def _():
            Md = _bdot(Bm_scr[...], D_ref[...])
            out_ref[0, :] = jnp.sum(orig_ref[...] * Md, axis=1)
            out_ref[1, :] = jnp.sum(origf_ref[...] * Md, axis=1)


def kernel(g, x, original, W_gcn, alphas, conv_w, conv_b, score_w, disc_w):
    Bg, N, _ = g.shape
    l_dim = W_gcn.shape[1]
    K = 64
    TILE = 512
    S = N // TILE

    w0T = conv_w[:, :, 0].T
    w1T = conv_w[:, :, 1].T
    cb_row = conv_b.reshape(1, l_dim)
    sw_col = score_w.reshape(l_dim, 1)
    orig_f = original[::-1]

    const = lambda *s: pl.BlockSpec(s, lambda b, t: (0,) * len(s))
    out2 = pl.pallas_call(
        _body,
        grid=(Bg, S),
        in_specs=[
            pl.BlockSpec((1, TILE, N), lambda b, t: (b, t, 0)),
            pl.BlockSpec((1, N, x.shape[2]), lambda b, t: (b, 0, 0)),
            const(Bg, l_dim),
            const(Bg, l_dim),
            const(W_gcn.shape[0], l_dim),
            const(1, alphas.shape[1]),
            const(l_dim, l_dim),
            const(l_dim, l_dim),
            const(1, l_dim),
            const(l_dim, 1),
            const(l_dim, l_dim),
        ],
        out_specs=pl.BlockSpec((2, Bg), lambda b, t: (0, 0)),
        out_shape=jax.ShapeDtypeStruct((2, Bg), _F32),
        scratch_shapes=[
            pltpu.VMEM((N, l_dim), _BF16),
            pltpu.VMEM((N, l_dim), _F32),
            pltpu.VMEM((N // 128, 128), _F32),
            pltpu.VMEM((K, N), _F32),
            pltpu.VMEM((Bg, l_dim), _F32),
        ],
        compiler_params=pltpu.CompilerParams(
            dimension_semantics=("arbitrary", "arbitrary"),
        ),
    )(g, x, original, orig_f, W_gcn, alphas, w0T, w1T, cb_row, sw_col, disc_w)

    return jnp.concatenate([out2[0], out2[1][::-1]], axis=0)


# speedup vs baseline: 1.0432x; 1.0278x over previous
import jax
import jax.numpy as jnp
from jax.experimental import pallas as pl
from jax.experimental.pallas import tpu as pltpu

_F32 = jnp.float32
_BF16 = jnp.bfloat16


def _bdot(a, b):
    return jnp.dot(a.astype(_BF16), b.astype(_BF16),
                   preferred_element_type=_F32)


def _body(g_ref, x_ref, orig_ref, origf_ref, W_ref, al_ref, w0T_ref, w1T_ref,
          cb_ref, sw_ref, D_ref, out_ref, xw_scr, e_scr, s_scr,
          Bm_scr):
    b = pl.program_id(0)
    t = pl.program_id(1)
    Bg = pl.num_programs(0)
    S = pl.num_programs(1)
    N, l_dim = e_scr.shape
    K = 64
    TILE = g_ref.shape[1]

    @pl.when(t == 0)
    def _():
        xw_scr[...] = _bdot(x_ref[0], W_ref[...]).astype(_BF16)

    gt = g_ref[0]
    deg = jnp.sum(gt, axis=1, keepdims=True)
    h = jnp.dot((gt / deg).astype(_BF16), xw_scr[...],
                preferred_element_type=_F32)
    hn = jnp.minimum(h, 0.0)
    u = jnp.exp(hn)
    em1 = jnp.where(u == 1.0, hn, (u - 1.0) * hn / jnp.log(u))
    e_scr[pl.ds(t * TILE, TILE), :] = jnp.where(h > 0, h, em1)

    @pl.when(t == S - 1)
    def _():
        e = e_scr[...]
        n_el = jnp.float32(N * l_dim)
        mu = jnp.sum(e) / n_el
        ec = e - mu
        var = jnp.sum(ec * ec) / n_el
        sig = jnp.sqrt(var + 1e-5)

        s_scr[...] = _bdot((e - mu) / sig, sw_ref[...]).reshape(N // 128, 128)

        flat_iota = (jax.lax.broadcasted_iota(jnp.int32, (N // 128, 128), 0)
                     * 128
                     + jax.lax.broadcasted_iota(jnp.int32, (N // 128, 128), 1))
        row_iota = jax.lax.broadcasted_iota(jnp.int32, (K, N), 1)
        lane_iota = jax.lax.broadcasted_iota(jnp.int32, (1, 128), 1)

        def tk_body(i, carry):
            s, idxs = carry
            m = jnp.max(s, keepdims=True)
            idx = jnp.min(jnp.where(s >= m, flat_iota, N), keepdims=True)
            idxs = jnp.where(lane_iota == i, idx, idxs)
            return jnp.where(flat_iota == idx, -jnp.inf, s), idxs

        idxs0 = jnp.zeros((1, 128), jnp.int32)
        _, idxs = jax.lax.fori_loop(0, K, tk_body, (s_scr[...], idxs0))

        idx_col = jnp.transpose(idxs)[:K]
        P = (row_iota == idx_col).astype(_F32)
        e_hi = e.astype(_BF16)
        e_r = e - e_hi.astype(_F32)
        e_mid = e_r.astype(_BF16)
        e_lo = (e_r - e_mid.astype(_F32)).astype(_BF16)
        Pb = P.astype(_BF16)
        picked = (jnp.dot(Pb, e_hi, preferred_element_type=_F32)
                  + (jnp.dot(Pb, e_mid, preferred_element_type=_F32)
                     + jnp.dot(Pb, e_lo, preferred_element_type=_F32)))
        sub = (picked - mu) / sig

        w0T = w0T_ref[...]
        w1T = w1T_ref[...]
        cb = cb_ref[...]

        def cnn(z):
            y0 = _bdot(z, w0T)
            y1 = _bdot(z, w1T)
            y = jnp.maximum(y0[:-1] + y1[1:] + cb, 0.0)
            return jnp.max(y, axis=0)

        a0 = al_ref[0, 0]
        a1 = al_ref[0, 1]
        a2 = al_ref[0, 2]
        am = jnp.maximum(a0, jnp.maximum(a1, a2))
        e0 = jnp.exp(a0 - am)
        e1 = jnp.exp(a1 - am)
        e2 = jnp.exp(a2 - am)
        inv_se = 1.0 / (e0 + e1 + e2)

        bvec = (e0 * cnn(sub) + e1 * cnn(jnp.maximum(sub, 0.0))
                + e2 * cnn(jnp.tanh(sub))) * inv_se
        Bm_scr[pl.ds(b, 1), :] = bvec.reshape(1, l_dim)

        @pl.when(b == Bg - 1)
        def _():
            Md = _bdot(Bm_scr[...], D_ref[...])
            out_ref[0, :] = jnp.sum(orig_ref[...] * Md, axis=1)
            out_ref[1, :] = jnp.sum(origf_ref[...] * Md, axis=1)


def kernel(g, x, original, W_gcn, alphas, conv_w, conv_b, score_w, disc_w):
    Bg, N, _ = g.shape
    l_dim = W_gcn.shape[1]
    K = 64
    TILE = 512
    S = N // TILE

    w0T = conv_w[:, :, 0].T
    w1T = conv_w[:, :, 1].T
    cb_row = conv_b.reshape(1, l_dim)
    sw_col = score_w.reshape(l_dim, 1)
    orig_f = original[::-1]

    const = lambda *s: pl.BlockSpec(s, lambda b, t: (0,) * len(s))
    out2 = pl.pallas_call(
        _body,
        grid=(Bg, S),
        in_specs=[
            pl.BlockSpec((1, TILE, N), lambda b, t: (b, t, 0)),
            pl.BlockSpec((1, N, x.shape[2]), lambda b, t: (b, 0, 0)),
            const(Bg, l_dim),
            const(Bg, l_dim),
            const(W_gcn.shape[0], l_dim),
            const(1, alphas.shape[1]),
            const(l_dim, l_dim),
            const(l_dim, l_dim),
            const(1, l_dim),
            const(l_dim, 1),
            const(l_dim, l_dim),
        ],
        out_specs=pl.BlockSpec((2, Bg), lambda b, t: (0, 0)),
        out_shape=jax.ShapeDtypeStruct((2, Bg), _F32),
        scratch_shapes=[
            pltpu.VMEM((N, l_dim), _BF16),
            pltpu.VMEM((N, l_dim), _F32),
            pltpu.VMEM((N // 128, 128), _F32),
            pltpu.VMEM((Bg, l_dim), _F32),
        ],
        compiler_params=pltpu.CompilerParams(
            dimension_semantics=("arbitrary", "arbitrary"),
        ),
    )(g, x, original, orig_f, W_gcn, alphas, w0T, w1T, cb_row, sw_col, disc_w)

    return jnp.concatenate([out2[0], out2[1][::-1]], axis=0)


# speedup vs baseline: 2.4631x; 2.3611x over previous
import jax
import jax.numpy as jnp
from jax.experimental import pallas as pl
from jax.experimental.pallas import tpu as pltpu

_F32 = jnp.float32
_BF16 = jnp.bfloat16


def _bdot(a, b):
    return jnp.dot(a.astype(_BF16), b.astype(_BF16),
                   preferred_element_type=_F32)


def _body(g_ref, x_ref, orig_ref, origf_ref, W_ref, al_ref, w0T_ref, w1T_ref,
          cb_ref, sw_ref, D_ref, out_ref, xw_scr, e_scr, s_scr, mu_scr,
          sg_scr):
    b = pl.program_id(0)
    t = pl.program_id(1)
    Bg = pl.num_programs(0)
    S = pl.num_programs(1)
    _, N, l_dim = e_scr.shape
    K = 64
    TILE = g_ref.shape[1]

    @pl.when(t == 0)
    def _():
        xw_scr[...] = _bdot(x_ref[0], W_ref[...]).astype(_BF16)

    gt = g_ref[0]
    deg = jnp.sum(gt, axis=1, keepdims=True)
    h = jnp.dot((gt / deg).astype(_BF16), xw_scr[...],
                preferred_element_type=_F32)
    hneg = jnp.minimum(h, 0.0)
    u = jnp.exp(hneg)
    em1 = jnp.where(u == 1.0, hneg, (u - 1.0) * hneg / jnp.log(u))
    e_scr[b, pl.ds(t * TILE, TILE), :] = jnp.where(h > 0, h, em1)

    @pl.when(t == S - 1)
    def _():
        e = e_scr[b]
        n_el = jnp.float32(N * l_dim)
        mu = jnp.sum(e) / n_el
        ec = e - mu
        var = jnp.sum(ec * ec) / n_el
        sig = jnp.sqrt(var + 1e-5)
        mu_scr[pl.ds(b, 1), :] = jnp.full((1, 128), mu, _F32)
        sg_scr[pl.ds(b, 1), :] = jnp.full((1, 128), sig, _F32)
        s_scr[b] = _bdot((e - mu) / sig, sw_ref[...]).reshape(N // 128, 128)

    @pl.when(jnp.logical_and(b == Bg - 1, t == S - 1))
    def _():
        s3 = s_scr[...]
        flat3 = (jax.lax.broadcasted_iota(jnp.int32, s3.shape, 1) * 128
                 + jax.lax.broadcasted_iota(jnp.int32, s3.shape, 2))
        lane3 = jax.lax.broadcasted_iota(jnp.int32, (Bg, 1, 128), 2)
        row_iota = jax.lax.broadcasted_iota(jnp.int32, (K, N), 1)

        def tk_body(i, carry):
            s, idxs = carry
            m12 = jnp.max(s, axis=2, keepdims=True)
            m3 = jnp.max(m12, axis=1, keepdims=True)
            cand = jnp.where(s >= m3, flat3, N)
            i12 = jnp.min(cand, axis=2, keepdims=True)
            idx3 = jnp.min(i12, axis=1, keepdims=True)
            idxs = jnp.where(lane3 == i, idx3, idxs)
            return jnp.where(flat3 == idx3, -jnp.inf, s), idxs

        idxs0 = jnp.zeros((Bg, 1, 128), jnp.int32)
        _, idxs = jax.lax.fori_loop(0, K, tk_body, (s3, idxs0))

        a0 = al_ref[0, 0]
        a1 = al_ref[0, 1]
        a2 = al_ref[0, 2]
        am = jnp.maximum(a0, jnp.maximum(a1, a2))
        e0 = jnp.exp(a0 - am)
        e1 = jnp.exp(a1 - am)
        e2 = jnp.exp(a2 - am)
        inv_se = 1.0 / (e0 + e1 + e2)

        w0T = w0T_ref[...]
        w1T = w1T_ref[...]
        cb = cb_ref[...]

        def cnn(z):
            y0 = _bdot(z, w0T)
            y1 = _bdot(z, w1T)
            y = jnp.maximum(y0[:-1] + y1[1:] + cb, 0.0)
            return jnp.max(y, axis=0)

        rows = []
        for gi in range(Bg):
            idx_col = jnp.transpose(idxs[gi])[:K]
            P = (row_iota == idx_col).astype(_BF16)
            eg = e_scr[gi]
            e_hi = eg.astype(_BF16)
            e_r = eg - e_hi.astype(_F32)
            e_mid = e_r.astype(_BF16)
            e_lo = (e_r - e_mid.astype(_F32)).astype(_BF16)
            picked = (jnp.dot(P, e_hi, preferred_element_type=_F32)
                      + (jnp.dot(P, e_mid, preferred_element_type=_F32)
                         + jnp.dot(P, e_lo, preferred_element_type=_F32)))
            mu_g = mu_scr[gi, 0]
            sg_g = sg_scr[gi, 0]
            sub = (picked - mu_g) / sg_g
            bvec = (e0 * cnn(sub) + e1 * cnn(jnp.maximum(sub, 0.0))
                    + e2 * cnn(jnp.tanh(sub))) * inv_se
            rows.append(bvec.reshape(1, l_dim))

        Bm = jnp.concatenate(rows, axis=0)
        Md = _bdot(Bm, D_ref[...])
        out_ref[0, :] = jnp.sum(orig_ref[...] * Md, axis=1)
        out_ref[1, :] = jnp.sum(origf_ref[...] * Md, axis=1)


def kernel(g, x, original, W_gcn, alphas, conv_w, conv_b, score_w, disc_w):
    Bg, N, _ = g.shape
    l_dim = W_gcn.shape[1]
    TILE = 512
    S = N // TILE

    w0T = conv_w[:, :, 0].T
    w1T = conv_w[:, :, 1].T
    cb_row = conv_b.reshape(1, l_dim)
    sw_col = score_w.reshape(l_dim, 1)
    orig_f = original[::-1]

    const = lambda *s: pl.BlockSpec(s, lambda b, t: (0,) * len(s))
    out2 = pl.pallas_call(
        _body,
        grid=(Bg, S),
        in_specs=[
            pl.BlockSpec((1, TILE, N), lambda b, t: (b, t, 0)),
            pl.BlockSpec((1, N, x.shape[2]), lambda b, t: (b, 0, 0)),
            const(Bg, l_dim),
            const(Bg, l_dim),
            const(W_gcn.shape[0], l_dim),
            const(1, alphas.shape[1]),
            const(l_dim, l_dim),
            const(l_dim, l_dim),
            const(1, l_dim),
            const(l_dim, 1),
            const(l_dim, l_dim),
        ],
        out_specs=pl.BlockSpec((2, Bg), lambda b, t: (0, 0)),
        out_shape=jax.ShapeDtypeStruct((2, Bg), _F32),
        scratch_shapes=[
            pltpu.VMEM((N, l_dim), _BF16),
            pltpu.VMEM((Bg, N, l_dim), _F32),
            pltpu.VMEM((Bg, N // 128, 128), _F32),
            pltpu.VMEM((Bg, 128), _F32),
            pltpu.VMEM((Bg, 128), _F32),
        ],
        compiler_params=pltpu.CompilerParams(
            dimension_semantics=("arbitrary", "arbitrary"),
        ),
    )(g, x, original, orig_f, W_gcn, alphas, w0T, w1T, cb_row, sw_col, disc_w)

    return jnp.concatenate([out2[0], out2[1][::-1]], axis=0)
